# baseline (speedup 1.0000x reference)
import functools

import numpy as np
import jax
import jax.numpy as jnp
from jax import lax
from jax.experimental import pallas as pl
from jax.experimental.pallas import tpu as pltpu

_THRESHOLD = 128
_GROUPS = 2


def _dft_mats(h: int, w: int):
    hh = np.arange(h)
    ah = np.exp(2j * np.pi * np.outer(hh, hh) / h) / h
    ww = np.arange(w)
    bw = np.exp(-2j * np.pi * np.outer(ww, ww) / w)
    br2 = np.zeros((w, 2 * w), np.float64)
    bi2 = np.zeros((w, 2 * w), np.float64)
    br2[:, 0::2] = bw.real
    br2[:, 1::2] = bw.imag
    bi2[:, 0::2] = -bw.imag
    bi2[:, 1::2] = bw.real
    half = np.zeros((8, 2 * w), np.float32)
    half[:, 0::2] = 0.5
    return (
        jnp.asarray(ah.real, jnp.float32),
        jnp.asarray(ah.imag, jnp.float32),
        jnp.asarray(ah.real + ah.imag, jnp.float32),
        jnp.asarray(br2, jnp.float32),
        jnp.asarray(bi2, jnp.float32),
        jnp.asarray(half, jnp.float32),
    )


def _kernel(xr_ref, xi_ref, ar_ref, ai_ref, as_ref, br2_ref, bi2_ref, half_ref, o_ref):
    g0r = xr_ref[0, 0]
    g0i = xi_ref[0, 0]
    g1r = xr_ref[0, 1]
    g1i = xi_ref[0, 1]

    e = jnp.sum(
        jnp.abs(jnp.abs(g0r) - jnp.abs(g1i)) + jnp.abs(jnp.abs(g1r) - jnp.abs(g0i)),
        axis=0,
    )

    w = e.shape[0]
    ecol = e[:, None]
    erow = e[None, :]
    lt = ecol < erow
    eq = ecol == erow
    iw = lax.broadcasted_iota(jnp.int32, (w, w), 0)
    jw = lax.broadcasted_iota(jnp.int32, (w, w), 1)
    before = iw < jw
    rank = jnp.sum(
        jnp.where(lt | (eq & before), jnp.int32(1), jnp.int32(0)), axis=0
    )

    jj = lax.broadcasted_iota(jnp.int32, (_THRESHOLD, w), 0)
    p = jnp.where(rank[None, :] == jj, jnp.float32(1.0), jnp.float32(0.0))

    mm = lambda a, b: jnp.dot(a, b, preferred_element_type=jnp.float32)

    ar = ar_ref[...]
    ai = ai_ref[...]
    asum = as_ref[...]
    csel_r = mm(p, br2_ref[...])
    csel_i = mm(p, bi2_ref[...])
    half = half_ref[0:1, :]

    compact = lambda m: lax.dot_general(
        m, p, (((1,), (1,)), ((), ())), preferred_element_type=jnp.float32
    )

    for c, (gr, gi) in enumerate(((g0r, g0i), (g1r, g1i))):
        gsr = compact(gr)
        gsi = compact(gi)
        m1 = mm(ar, gsr)
        m2 = mm(ai, gsi)
        m3 = mm(asum, gsr + gsi)
        tr = m1 - m2
        ti = m3 - m1 - m2
        o_ref[0, c] = mm(tr, csel_r) + mm(ti, csel_i) + half


@functools.partial(jax.jit, static_argnums=())
def kernel(Inp_AD_C_real, Inp_AD_C_imag):
    b, c, h, w = Inp_AD_C_real.shape
    ar, ai, asum, br2, bi2, half = _dft_mats(h, w)
    cg = c // _GROUPS

    x_spec = pl.BlockSpec((1, cg, h, w), lambda ib, ig: (ib, ig, 0, 0))
    m_spec = pl.BlockSpec((h, w), lambda ib, ig: (0, 0))
    m2_spec = pl.BlockSpec((w, 2 * w), lambda ib, ig: (0, 0))
    h_spec = pl.BlockSpec((8, 2 * w), lambda ib, ig: (0, 0))
    o_spec = pl.BlockSpec((1, cg, h, 2 * w), lambda ib, ig: (ib, ig, 0, 0))
    out = pl.pallas_call(
        _kernel,
        grid=(b, _GROUPS),
        in_specs=[x_spec, x_spec, m_spec, m_spec, m_spec, m2_spec, m2_spec, h_spec],
        out_specs=o_spec,
        out_shape=jax.ShapeDtypeStruct((b, c, h, 2 * w), jnp.float32),
        compiler_params=pltpu.CompilerParams(
            dimension_semantics=("parallel", "parallel"),
        ),
    )(Inp_AD_C_real, Inp_AD_C_imag, ar, ai, asum, br2, bi2, half)
    return out.reshape(b, c, h, w, 2).view(jnp.complex64).reshape(b, c, h, w)
